# baseline (speedup 1.0000x reference)
import functools

import jax
import jax.numpy as jnp
from jax import lax
from jax.experimental import pallas as pl
from jax.experimental.pallas import tpu as pltpu
from jax.experimental.pallas import tpu_sc as plsc

_N = 10000
_G = 100
_NPG = 100
_E_PER = 3200
_D = 128
_K = 800
_OUT = 10
_GB = 20
_SORT = 4096
_NEG = -3.0e38
_GSC = 128
_WPG = 4


def _bitonic_desc(x):
    rows, n = x.shape
    lane = lax.broadcasted_iota(jnp.int32, (1, n), 1)
    k = 2
    while k <= n:
        jd = k // 2
        while jd >= 1:
            up = jnp.concatenate([x[:, jd:], x[:, :jd]], axis=1)
            dn = jnp.concatenate([x[:, -jd:], x[:, :-jd]], axis=1)
            low = (lane & jd) == 0
            part = jnp.where(low, up, dn)
            asc_blk = (lane & k) == 0
            keep_max = asc_blk == low
            x = jnp.where(keep_max, jnp.maximum(x, part),
                          jnp.minimum(x, part))
            jd //= 2
        k *= 2
    return x


def _mlp_sort_body(row_ref, col_ref, h_ref, w1_ref, b1_ref, w2_ref, b2_ref,
                   sg_ref, cw_ref, sw_ref, tr_ref, nr_ref):
    pid = pl.program_id(0)
    b1 = b1_ref[...]
    w2b = w2_ref[...].astype(jnp.bfloat16).astype(jnp.float32)
    b2 = b2_ref[0, 0]
    iota = lax.broadcasted_iota(jnp.int32, (_E_PER, _NPG), 1)
    outs = []
    for j in range(_GB):
        hg = h_ref[j * _NPG:(j + 1) * _NPG, :]
        h_hi = hg.astype(jnp.bfloat16)
        base = (pid * _GB + j) * _NPG
        r = row_ref[0, j, :] - base
        c = col_ref[0, j, :] - base
        ohr = (lax.broadcast_in_dim(r, (_E_PER, _NPG), (0,)) == iota
               ).astype(jnp.bfloat16)
        ohc = (lax.broadcast_in_dim(c, (_E_PER, _NPG), (0,)) == iota
               ).astype(jnp.bfloat16)
        hr = jnp.dot(ohr, h_hi, preferred_element_type=jnp.float32)
        hc = jnp.dot(ohc, h_hi, preferred_element_type=jnp.float32)
        hrc = jnp.concatenate([hr, hc], axis=1)
        z = jnp.maximum(
            jnp.dot(hrc, w1_ref[...], preferred_element_type=jnp.float32)
            + b1, 0.0)
        zb16 = z.astype(jnp.bfloat16).astype(jnp.float32)
        w = jnp.sum(zb16 * w2b, axis=1) + b2
        outs.append(w)
    sg = jnp.stack(outs, axis=0)
    sg_ref[0] = sg
    pad = jnp.full((_GB, _SORT - _E_PER), _NEG, jnp.float32)
    s = _bitonic_desc(jnp.concatenate([sg, pad], axis=1))
    cw_ref[0] = s[:, :_K]
    sw_ref[0] = -s[:, _K:_E_PER]
    t = s[:, _K - 1:_K]
    cnt_gt = jnp.sum((sg > t).astype(jnp.float32), axis=1, keepdims=True)
    tr_ref[0] = jnp.broadcast_to(t, (_GB, 16))
    nr_ref[0] = jnp.broadcast_to(_K - cnt_gt, (_GB, 16))


def _sc_body(sg_hbm, row_hbm, t_hbm, need_hbm, h_hbm, wct_hbm, bc_hbm,
             out_hbm, sg_v, row_v, t_v, need_v, h_v, wct_v, bc_v, u_v,
             out_v):
    wid = lax.axis_index("s") * 2 + lax.axis_index("c")
    lane = lax.iota(jnp.int32, 16)
    zero16 = jnp.zeros((16,), jnp.float32)
    pltpu.sync_copy(wct_hbm, wct_v)
    pltpu.sync_copy(bc_hbm, bc_v)
    for gi in range(_WPG):
        g = wid * _WPG + gi

        @pl.when(g < _G)
        def _():
            pltpu.sync_copy(sg_hbm.at[g], sg_v)
            pltpu.sync_copy(row_hbm.at[g], row_v)
            pltpu.sync_copy(t_hbm.at[g], t_v)
            pltpu.sync_copy(need_hbm.at[g], need_v)
            pltpu.sync_copy(h_hbm.at[g], h_v)
            t = t_v[...]
            need = need_v[...]
            base = g * _NPG

            def zbody(i, c):
                u_v[pl.ds(i * 16, 16)] = zero16
                return c
            lax.fori_loop(0, _NPG, zbody, 0, unroll=False)

            def mbody(i, carry):
                v = sg_v[pl.ds(i * 16, 16)]
                r = row_v[pl.ds(i * 16, 16)] - base
                eq = v == t
                gt = v > t
                cs = plsc.cumsum(jnp.where(eq, 1.0, 0.0))
                pos = carry + cs
                m = gt | (eq & (pos <= need))
                m = m & (r >= 0) & (r < _NPG)
                plsc.addupdate_scatter(u_v, [r * 16 + lane], v, mask=m)
                return jnp.max(pos)
            lax.fori_loop(0, _E_PER // 16, mbody, jnp.float32(0.0),
                          unroll=False)

            def pbody(n, acc):
                un = jnp.sum(u_v[pl.ds(n * 16, 16)])
                return tuple(
                    acc[dc] + un * h_v[pl.ds(n * _D + dc * 16, 16)]
                    for dc in range(8))
            pooled = lax.fori_loop(0, _NPG, pbody,
                                   tuple(zero16 for _ in range(8)),
                                   unroll=False)
            pred = bc_v[...]
            for dc in range(8):
                pu = plsc.bitcast(pooled[dc] * (1.0 / _K), jnp.uint32)
                lsb = (pu >> 16) & jnp.uint32(1)
                pu = (pu + jnp.uint32(0x7FFF) + lsb) & jnp.uint32(0xFFFF0000)
                pv = plsc.bitcast(pu, jnp.float32)
                for l in range(16):
                    coef = jnp.sum(jnp.where(lane == l, pv, zero16))
                    pred = pred + coef * wct_v[pl.ds((dc * 16 + l) * 16, 16)]
            out_v[...] = pred
            pltpu.sync_copy(out_v, out_hbm.at[g])


def _sc_select_pool(sg_sc, row_sc, t_sc, nr_sc, h3, wct, bcp):
    mesh = plsc.VectorSubcoreMesh(core_axis_name="c", subcore_axis_name="s")
    f = pl.kernel(
        _sc_body,
        jax.ShapeDtypeStruct((_G, 16), jnp.float32),
        mesh=mesh,
        scratch_types=[
            pltpu.VMEM((_E_PER,), jnp.float32),
            pltpu.VMEM((_E_PER,), jnp.int32),
            pltpu.VMEM((16,), jnp.float32),
            pltpu.VMEM((16,), jnp.float32),
            pltpu.VMEM((_NPG * _D,), jnp.float32),
            pltpu.VMEM((_D * 16,), jnp.float32),
            pltpu.VMEM((16,), jnp.float32),
            pltpu.VMEM((_NPG * 16,), jnp.float32),
            pltpu.VMEM((16,), jnp.float32),
        ],
        compiler_params=pltpu.CompilerParams(needs_layout_passes=False),
    )
    return f(sg_sc, row_sc, t_sc, nr_sc, h3, wct, bcp)


def kernel(h, edge_index, W1, b1, W2, b2, Wc, bc):
    f32 = jnp.float32
    row = edge_index[0].astype(jnp.int32).reshape(_G // _GB, _GB, _E_PER)
    col = edge_index[1].astype(jnp.int32).reshape(_G // _GB, _GB, _E_PER)
    b1r = b1.reshape(1, 4 * _D)
    w2r = W2.reshape(1, 4 * _D)
    b2r = b2.reshape(1, 1)

    sg3, cw_p, sw_p, t_p, nr_p = pl.pallas_call(
        _mlp_sort_body,
        grid=(_G // _GB,),
        in_specs=[
            pl.BlockSpec((1, _GB, _E_PER), lambda i: (i, 0, 0)),
            pl.BlockSpec((1, _GB, _E_PER), lambda i: (i, 0, 0)),
            pl.BlockSpec((_GB * _NPG, _D), lambda i: (i, 0)),
            pl.BlockSpec((2 * _D, 4 * _D), lambda i: (0, 0)),
            pl.BlockSpec((1, 4 * _D), lambda i: (0, 0)),
            pl.BlockSpec((1, 4 * _D), lambda i: (0, 0)),
            pl.BlockSpec(memory_space=pltpu.SMEM),
        ],
        out_specs=[
            pl.BlockSpec((1, _GB, _E_PER), lambda i: (i, 0, 0)),
            pl.BlockSpec((1, _GB, _K), lambda i: (i, 0, 0)),
            pl.BlockSpec((1, _GB, _E_PER - _K), lambda i: (i, 0, 0)),
            pl.BlockSpec((1, _GB, 16), lambda i: (i, 0, 0)),
            pl.BlockSpec((1, _GB, 16), lambda i: (i, 0, 0)),
        ],
        out_shape=[
            jax.ShapeDtypeStruct((_G // _GB, _GB, _E_PER), f32),
            jax.ShapeDtypeStruct((_G // _GB, _GB, _K), f32),
            jax.ShapeDtypeStruct((_G // _GB, _GB, _E_PER - _K), f32),
            jax.ShapeDtypeStruct((_G // _GB, _GB, 16), f32),
            jax.ShapeDtypeStruct((_G // _GB, _GB, 16), f32),
        ],
        compiler_params=pltpu.CompilerParams(
            dimension_semantics=("arbitrary",)),
    )(row, col, h, W1, b1r, w2r, b2r)

    sg_sc = sg3.reshape(_G, _E_PER)
    row_sc = row.reshape(_G, _E_PER)
    t_sc = t_p.reshape(_G, 16)
    nr_sc = nr_p.reshape(_G, 16)
    h2 = h.reshape(_G, _NPG * _D)
    wct = jnp.pad(Wc.astype(jnp.bfloat16).astype(f32),
                  ((0, 0), (0, 16 - _OUT))).reshape(_D * 16)
    bcp = jnp.pad(bc, ((0, 16 - _OUT),))

    pred_sc = _sc_select_pool(sg_sc, row_sc, t_sc, nr_sc, h2, wct, bcp)

    causal_pred = pred_sc[:, :_OUT]
    causal_w = cw_p.reshape(_G, _K)
    spu_w = sw_p.reshape(_G, _E_PER - _K)
    return (causal_pred, causal_w, spu_w)


# speedup vs baseline: 1.2767x; 1.2767x over previous
import functools

import jax
import jax.numpy as jnp
from jax import lax
from jax.experimental import pallas as pl
from jax.experimental.pallas import tpu as pltpu
from jax.experimental.pallas import tpu_sc as plsc

_N = 10000
_G = 100
_NPG = 100
_E_PER = 3200
_D = 128
_K = 800
_OUT = 10
_GB1 = 4
_GB2 = 20
_SORT = 4096
_NEG = -3.0e38
_GSC = 128
_WPG = 4


def _bitonic_desc(x):
    rows, n = x.shape
    lane = lax.broadcasted_iota(jnp.int32, (1, n), 1)
    k = 2
    while k <= n:
        jd = k // 2
        while jd >= 1:
            up = jnp.concatenate([x[:, jd:], x[:, :jd]], axis=1)
            dn = jnp.concatenate([x[:, -jd:], x[:, :-jd]], axis=1)
            low = (lane & jd) == 0
            part = jnp.where(low, up, dn)
            asc_blk = (lane & k) == 0
            keep_max = asc_blk == low
            x = jnp.where(keep_max, jnp.maximum(x, part),
                          jnp.minimum(x, part))
            jd //= 2
        k *= 2
    return x


def _edge_mlp_body(row_ref, col_ref, h_ref, w1_ref, b1_ref, w2_ref, b2_ref,
                   out_ref):
    pid = pl.program_id(0)
    b1 = b1_ref[...]
    w2b = w2_ref[...].astype(jnp.bfloat16).astype(jnp.float32)
    b2 = b2_ref[0, 0]
    iota = lax.broadcasted_iota(jnp.int32, (_E_PER, _NPG), 1)
    outs = []
    for j in range(_GB1):
        hg = h_ref[j * _NPG:(j + 1) * _NPG, :]
        h_hi = hg.astype(jnp.bfloat16)
        base = (pid * _GB1 + j) * _NPG
        r = row_ref[0, j, :] - base
        c = col_ref[0, j, :] - base
        ohr = (lax.broadcast_in_dim(r, (_E_PER, _NPG), (0,)) == iota
               ).astype(jnp.bfloat16)
        ohc = (lax.broadcast_in_dim(c, (_E_PER, _NPG), (0,)) == iota
               ).astype(jnp.bfloat16)
        hr = jnp.dot(ohr, h_hi, preferred_element_type=jnp.float32)
        hc = jnp.dot(ohc, h_hi, preferred_element_type=jnp.float32)
        hrc = jnp.concatenate([hr, hc], axis=1)
        z = jnp.maximum(
            jnp.dot(hrc, w1_ref[...], preferred_element_type=jnp.float32)
            + b1, 0.0)
        zb16 = z.astype(jnp.bfloat16).astype(jnp.float32)
        w = jnp.sum(zb16 * w2b, axis=1) + b2
        outs.append(w)
    out_ref[0] = jnp.stack(outs, axis=0)


def _sort_body(sg_ref, cw_ref, sw_ref, tr_ref, nr_ref):
    sg = sg_ref[0]
    pad = jnp.full((_GB2, _SORT - _E_PER), _NEG, jnp.float32)
    s = _bitonic_desc(jnp.concatenate([sg, pad], axis=1))
    cw_ref[0] = s[:, :_K]
    sw_ref[0] = -s[:, _K:_E_PER]
    t = s[:, _K - 1:_K]
    cnt_gt = jnp.sum((sg > t).astype(jnp.float32), axis=1, keepdims=True)
    tr_ref[0] = jnp.broadcast_to(t, (_GB2, 16))
    nr_ref[0] = jnp.broadcast_to(_K - cnt_gt, (_GB2, 16))


def _sc_body(sg_hbm, row_hbm, t_hbm, need_hbm, h_hbm, wct_hbm, bc_hbm,
             out_hbm, sg_v, row_v, t_v, need_v, h_v, wct_v, bc_v, u_v,
             out_v):
    wid = lax.axis_index("s") * 2 + lax.axis_index("c")
    lane = lax.iota(jnp.int32, 16)
    zero16 = jnp.zeros((16,), jnp.float32)
    pltpu.sync_copy(wct_hbm, wct_v)
    pltpu.sync_copy(bc_hbm, bc_v)
    for gi in range(_WPG):
        g = wid * _WPG + gi

        @pl.when(g < _G)
        def _():
            pltpu.sync_copy(sg_hbm.at[g], sg_v)
            pltpu.sync_copy(row_hbm.at[g], row_v)
            pltpu.sync_copy(t_hbm.at[g], t_v)
            pltpu.sync_copy(need_hbm.at[g], need_v)
            pltpu.sync_copy(h_hbm.at[g], h_v)
            t = t_v[...]
            need = need_v[...]
            base = g * _NPG

            def zbody(i, c):
                u_v[pl.ds(i * 16, 16)] = zero16
                return c
            lax.fori_loop(0, _NPG, zbody, 0, unroll=False)

            def mbody(i, carry):
                v = sg_v[pl.ds(i * 16, 16)]
                r = row_v[pl.ds(i * 16, 16)] - base
                eq = v == t
                gt = v > t
                cs = plsc.cumsum(jnp.where(eq, 1.0, 0.0))
                pos = carry + cs
                m = gt | (eq & (pos <= need))
                m = m & (r >= 0) & (r < _NPG)
                plsc.addupdate_scatter(u_v, [r * 16 + lane], v, mask=m)
                return jnp.max(pos)
            lax.fori_loop(0, _E_PER // 16, mbody, jnp.float32(0.0),
                          unroll=False)

            def pbody(n, acc):
                un = jnp.sum(u_v[pl.ds(n * 16, 16)])
                return tuple(
                    acc[dc] + un * h_v[pl.ds(n * _D + dc * 16, 16)]
                    for dc in range(8))
            pooled = lax.fori_loop(0, _NPG, pbody,
                                   tuple(zero16 for _ in range(8)),
                                   unroll=False)
            pred = bc_v[...]
            for dc in range(8):
                pu = plsc.bitcast(pooled[dc] * (1.0 / _K), jnp.uint32)
                lsb = (pu >> 16) & jnp.uint32(1)
                pu = (pu + jnp.uint32(0x7FFF) + lsb) & jnp.uint32(0xFFFF0000)
                pv = plsc.bitcast(pu, jnp.float32)
                for l in range(16):
                    coef = jnp.sum(jnp.where(lane == l, pv, zero16))
                    pred = pred + coef * wct_v[pl.ds((dc * 16 + l) * 16, 16)]
            out_v[...] = pred
            pltpu.sync_copy(out_v, out_hbm.at[g])


def _sc_select_pool(sg_sc, row_sc, t_sc, nr_sc, h3, wct, bcp):
    mesh = plsc.VectorSubcoreMesh(core_axis_name="c", subcore_axis_name="s")
    f = pl.kernel(
        _sc_body,
        jax.ShapeDtypeStruct((_G, 16), jnp.float32),
        mesh=mesh,
        scratch_types=[
            pltpu.VMEM((_E_PER,), jnp.float32),
            pltpu.VMEM((_E_PER,), jnp.int32),
            pltpu.VMEM((16,), jnp.float32),
            pltpu.VMEM((16,), jnp.float32),
            pltpu.VMEM((_NPG * _D,), jnp.float32),
            pltpu.VMEM((_D * 16,), jnp.float32),
            pltpu.VMEM((16,), jnp.float32),
            pltpu.VMEM((_NPG * 16,), jnp.float32),
            pltpu.VMEM((16,), jnp.float32),
        ],
        compiler_params=pltpu.CompilerParams(needs_layout_passes=False),
    )
    return f(sg_sc, row_sc, t_sc, nr_sc, h3, wct, bcp)


def kernel(h, edge_index, W1, b1, W2, b2, Wc, bc):
    f32 = jnp.float32
    row = edge_index[0].astype(jnp.int32).reshape(_G // _GB1, _GB1, _E_PER)
    col = edge_index[1].astype(jnp.int32).reshape(_G // _GB1, _GB1, _E_PER)
    b1r = b1.reshape(1, 4 * _D)
    w2r = W2.reshape(1, 4 * _D)
    b2r = b2.reshape(1, 1)

    sg3 = pl.pallas_call(
        _edge_mlp_body,
        grid=(_G // _GB1,),
        in_specs=[
            pl.BlockSpec((1, _GB1, _E_PER), lambda i: (i, 0, 0)),
            pl.BlockSpec((1, _GB1, _E_PER), lambda i: (i, 0, 0)),
            pl.BlockSpec((_GB1 * _NPG, _D), lambda i: (i, 0)),
            pl.BlockSpec((2 * _D, 4 * _D), lambda i: (0, 0)),
            pl.BlockSpec((1, 4 * _D), lambda i: (0, 0)),
            pl.BlockSpec((1, 4 * _D), lambda i: (0, 0)),
            pl.BlockSpec(memory_space=pltpu.SMEM),
        ],
        out_specs=pl.BlockSpec((1, _GB1, _E_PER), lambda i: (i, 0, 0)),
        out_shape=jax.ShapeDtypeStruct((_G // _GB1, _GB1, _E_PER), f32),
        compiler_params=pltpu.CompilerParams(
            dimension_semantics=("arbitrary",)),
    )(row, col, h, W1, b1r, w2r, b2r)

    sgp = sg3.reshape(_G // _GB2, _GB2, _E_PER)

    cw_p, sw_p, t_p, nr_p = pl.pallas_call(
        _sort_body,
        grid=(_G // _GB2,),
        in_specs=[
            pl.BlockSpec((1, _GB2, _E_PER), lambda i: (i, 0, 0)),
        ],
        out_specs=[
            pl.BlockSpec((1, _GB2, _K), lambda i: (i, 0, 0)),
            pl.BlockSpec((1, _GB2, _E_PER - _K), lambda i: (i, 0, 0)),
            pl.BlockSpec((1, _GB2, 16), lambda i: (i, 0, 0)),
            pl.BlockSpec((1, _GB2, 16), lambda i: (i, 0, 0)),
        ],
        out_shape=[
            jax.ShapeDtypeStruct((_G // _GB2, _GB2, _K), f32),
            jax.ShapeDtypeStruct((_G // _GB2, _GB2, _E_PER - _K), f32),
            jax.ShapeDtypeStruct((_G // _GB2, _GB2, 16), f32),
            jax.ShapeDtypeStruct((_G // _GB2, _GB2, 16), f32),
        ],
        compiler_params=pltpu.CompilerParams(
            dimension_semantics=("arbitrary",)),
    )(sgp)

    sg_sc = sg3.reshape(_G, _E_PER)
    row_sc = row.reshape(_G, _E_PER)
    t_sc = t_p.reshape(_G, 16)
    nr_sc = nr_p.reshape(_G, 16)
    h2 = h.reshape(_G, _NPG * _D)
    wct = jnp.pad(Wc.astype(jnp.bfloat16).astype(f32),
                  ((0, 0), (0, 16 - _OUT))).reshape(_D * 16)
    bcp = jnp.pad(bc, ((0, 16 - _OUT),))

    pred_sc = _sc_select_pool(sg_sc, row_sc, t_sc, nr_sc, h2, wct, bcp)

    causal_pred = pred_sc[:, :_OUT]
    causal_w = cw_p.reshape(_G, _K)
    spu_w = sw_p.reshape(_G, _E_PER - _K)
    return (causal_pred, causal_w, spu_w)
